# baseline (speedup 1.0000x reference)
import functools

import jax
import jax.numpy as jnp
from jax import lax
from jax.experimental import pallas as pl
from jax.experimental.pallas import tpu as pltpu
from jax.experimental.pallas import tpu_sc as plsc

B = 4096
D = 64
K = 8
N_REL = 32
WINDOW = 128


def _sc_gather_multi(pairs):
    mesh = plsc.VectorSubcoreMesh(core_axis_name="core", subcore_axis_name="subcore")
    out_types = [jax.ShapeDtypeStruct((idx.shape[0], t.shape[1]), t.dtype)
                 for t, idx in pairs]

    @functools.partial(
        pl.kernel,
        out_type=out_types,
        mesh=mesh,
        compiler_params=pltpu.CompilerParams(use_tc_tiling_on_sc=False),
    )
    def k(*refs):
        ins = refs[:2 * len(pairs)]
        outs = refs[2 * len(pairs):]
        for p, (t, idx) in enumerate(pairs):
            tab_hbm, i_hbm, o_hbm = ins[2 * p], ins[2 * p + 1], outs[p]
            n = idx.shape[0]
            vdim = t.shape[1]

            def body(i_vmem, o_vmem, tab=tab_hbm):
                pltpu.sync_copy(tab.at[i_vmem.at[0]], o_vmem)

            pltpu.emit_pipeline(
                body,
                grid=(n // WINDOW,),
                in_specs=[pl.BlockSpec((1, WINDOW), lambda i: (0, i))],
                out_specs=[pl.BlockSpec((WINDOW, vdim), lambda i: (i, 0))],
                core_axis_name=("core", "subcore"),
                dimension_semantics=(pltpu.PARALLEL,),
            )(i_hbm, o_hbm)

    flat = []
    for t, idx in pairs:
        flat.extend([t, idx.reshape(1, idx.shape[0])])
    outs = k(*flat)
    return outs if isinstance(outs, (list, tuple)) else [outs]


BB = 512


def _tc_dense(u, uid, evs3, r0, r1, rel, W0, b0, W1, b1, Bh=B):
    nblk = Bh // BB
    BP = BB * 73 // 16

    def body(u_ref, uid_ref, evs_ref, r0_ref, r1_ref, rel_ref,
             w0_ref, b0_ref, w1_ref, b1_ref, out_ref):
        hp = lambda a, bm: lax.dot_general(
            a, bm, (((1,), (1,)), ((), ())),
            preferred_element_type=jnp.float32)

        x2 = evs_ref[...].reshape(BB * 73 // 2, 128)
        ev2s = [x2[j * BB * 8:(j + 1) * BB * 8, :] for j in range(4)]
        ev1s = [x2[BB * 32 + j * BB:BB * 32 + (j + 1) * BB, :] for j in range(4)]
        ev0p = x2[BB * 36:BB * 36 + BB // 2, :]

        u128 = u_ref[...]
        odd = (uid_ref[...] & 1) == 1
        uu = jnp.where(odd, u128[:, D:], u128[:, :D])
        relm = rel_ref[...]
        P = hp(uu, relm) * (1.0 / D)
        E = jnp.exp(P)
        r0v = r0_ref[...]
        r1v = r1_ref[...]
        e0 = jnp.take_along_axis(E, r0v, axis=1)
        w0 = e0 / jnp.sum(e0, axis=1, keepdims=True)
        E2 = jnp.broadcast_to(E[:, None, :], (BB, K, N_REL)).reshape(BB * K, N_REL)
        e1v = jnp.take_along_axis(E2, r1v, axis=1)
        w1 = e1v / jnp.sum(e1v, axis=1, keepdims=True)

        li8 = lax.broadcasted_iota(jnp.int32, (K, 128), 1)
        ri8 = lax.broadcasted_iota(jnp.int32, (K, 128), 0)

        def pair_w(w, j, n):
            sel = (ri8 == jnp.where(li8 < D, 2 * j, 2 * j + 1)).astype(jnp.float32)
            return lax.dot_general(w, sel, (((1,), (0,)), ((), ())),
                                   preferred_element_type=jnp.float32)

        Rm = pair_w(w1, 0, BB * 8) * ev2s[0]
        for j in range(1, 4):
            Rm = Rm + pair_w(w1, j, BB * 8) * ev2s[j]

        Wm0 = w0_ref[...]
        bb0 = b0_ref[...]
        Wm1 = w1_ref[...]
        bb1 = b1_ref[...]
        z = jnp.zeros((D, D), jnp.float32)
        W0cat = jnp.concatenate([Wm0, Wm0], axis=1) * 0.125
        Y = hp(Rm, W0cat)
        Y3 = Y.reshape(BB, 8, D)
        BD0 = jnp.concatenate(
            [jnp.concatenate([Wm0, z], axis=1),
             jnp.concatenate([z, Wm0], axis=1)], axis=0)
        b0p = jnp.concatenate([bb0, bb0], axis=1)

        S3 = pair_w(w0, 0, BB) * ev1s[0]
        T = None
        h1s = []
        for j in range(4):
            pyj = jnp.concatenate([Y3[:, 2 * j, :], Y3[:, 2 * j + 1, :]], axis=1)
            h1j = jax.nn.relu(hp(ev1s[j], BD0) + pyj + b0p)
            h1s.append(h1j)
            tj = pair_w(w0, j, BB) * h1j
            T = tj if T is None else T + tj
            if j > 0:
                S3 = S3 + pair_w(w0, j, BB) * ev1s[j]
        aggf = (T[:, :D] + T[:, D:]) * 0.125
        agg0 = (S3[:, :D] + S3[:, D:]) * 0.125

        ev0 = jnp.stack([ev0p[:, :D], ev0p[:, D:]],
                        axis=1).reshape(BB, D)
        h0 = jax.nn.relu(hp(ev0 + agg0, Wm0) + bb0)
        i_emb = jnp.tanh(hp(h0 + aggf, Wm1) + bb1)
        out_ref[...] = jnp.sum(uu * i_emb, axis=1, keepdims=True)

    out = pl.pallas_call(
        body,
        grid=(nblk,),
        in_specs=[
            pl.BlockSpec((BB, 128), lambda i: (i, 0)),
            pl.BlockSpec((BB, 1), lambda i: (i, 0)),
            pl.BlockSpec((BP, 8, 128), lambda i: (i, 0, 0)),
            pl.BlockSpec((BB, K), lambda i: (i, 0)),
            pl.BlockSpec((BB * 8, K), lambda i: (i, 0)),
            pl.BlockSpec((N_REL, D), lambda i: (0, 0)),
            pl.BlockSpec((D, D), lambda i: (0, 0)),
            pl.BlockSpec((1, D), lambda i: (0, 0)),
            pl.BlockSpec((D, D), lambda i: (0, 0)),
            pl.BlockSpec((1, D), lambda i: (0, 0)),
        ],
        out_specs=pl.BlockSpec((BB, 1), lambda i: (i, 0)),
        out_shape=jax.ShapeDtypeStruct((Bh, 1), jnp.float32),
    )(u, uid, evs3, r0, r1, rel, W0, b0, W1, b1)
    return out.reshape(Bh)


def kernel(user_ids, item_ids, adj_entity, adj_relation, user_emb,
           entity_emb, relation_emb, W0, b0, W1, b1):
    item_ids = item_ids.astype(jnp.int32)
    user_ids = user_ids.astype(jnp.int32)
    fused_adj = jnp.concatenate(
        [adj_entity.astype(jnp.int32), adj_relation.astype(jnp.int32)], axis=1)

    user_pairs = user_emb.reshape(user_emb.shape[0] // 2, 2 * D)
    er1, u = _sc_gather_multi(
        [(fused_adj, item_ids), (user_pairs, user_ids >> 1)])
    e1 = er1[:, :K].reshape(-1)
    r0 = er1[:, K:]

    er2, = _sc_gather_multi([(fused_adj, e1)])
    e2 = er2[:, :K].reshape(-1)
    r1 = er2[:, K:]

    nblk = B // BB
    e2o = e2.reshape(nblk, BB, 8, 4, 2).transpose(0, 3, 1, 2, 4).reshape(nblk, BB * 64)
    e1o = e1.reshape(nblk, BB, 4, 2).transpose(0, 2, 1, 3).reshape(nblk, BB * 8)
    all_idx = jnp.concatenate(
        [e2o, e1o, item_ids.reshape(nblk, BB)], axis=1)

    Bh = B // 8
    nh = nblk // 8
    uid = user_ids.reshape(B, 1)
    outs = []
    for h in range(8):
        idx_h = all_idx[h * nh:(h + 1) * nh].reshape(-1)
        evs, = _sc_gather_multi([(entity_emb, idx_h)])
        evs3 = evs.reshape(Bh * 73 // 16, 8, 128)
        sl = slice(h * Bh, (h + 1) * Bh)
        sl8 = slice(h * Bh * 8, (h + 1) * Bh * 8)
        outs.append(_tc_dense(u[sl], uid[sl], evs3, r0[sl], r1[sl8],
                              relation_emb, W0, b0.reshape(1, D), W1,
                              b1.reshape(1, D), Bh=Bh))
    return jnp.concatenate(outs)


# speedup vs baseline: 1.0738x; 1.0738x over previous
import functools

import jax
import jax.numpy as jnp
from jax import lax
from jax.experimental import pallas as pl
from jax.experimental.pallas import tpu as pltpu
from jax.experimental.pallas import tpu_sc as plsc

B = 4096
D = 64
K = 8
N_REL = 32
WINDOW = 128


def _sc_gather_multi(pairs):
    mesh = plsc.VectorSubcoreMesh(core_axis_name="core", subcore_axis_name="subcore")
    out_types = [jax.ShapeDtypeStruct((idx.shape[0], t.shape[1]), t.dtype)
                 for t, idx in pairs]

    @functools.partial(
        pl.kernel,
        out_type=out_types,
        mesh=mesh,
        compiler_params=pltpu.CompilerParams(use_tc_tiling_on_sc=False),
    )
    def k(*refs):
        ins = refs[:2 * len(pairs)]
        outs = refs[2 * len(pairs):]
        for p, (t, idx) in enumerate(pairs):
            tab_hbm, i_hbm, o_hbm = ins[2 * p], ins[2 * p + 1], outs[p]
            n = idx.shape[0]
            vdim = t.shape[1]

            def body(i_vmem, o_vmem, tab=tab_hbm):
                pltpu.sync_copy(tab.at[i_vmem.at[0]], o_vmem)

            pltpu.emit_pipeline(
                body,
                grid=(n // WINDOW,),
                in_specs=[pl.BlockSpec((1, WINDOW), lambda i: (0, i))],
                out_specs=[pl.BlockSpec((WINDOW, vdim), lambda i: (i, 0))],
                core_axis_name=("core", "subcore"),
                dimension_semantics=(pltpu.PARALLEL,),
            )(i_hbm, o_hbm)

    flat = []
    for t, idx in pairs:
        flat.extend([t, idx.reshape(1, idx.shape[0])])
    outs = k(*flat)
    return outs if isinstance(outs, (list, tuple)) else [outs]


BB = 512


def _tc_dense(u, uid, evs3, r0, r1, rel, W0, b0, W1, b1, Bh=B):
    nblk = Bh // BB
    BP = BB * 73 // 16

    def body(u_ref, uid_ref, evs_ref, r0_ref, r1_ref, rel_ref,
             w0_ref, b0_ref, w1_ref, b1_ref, out_ref):
        hp = lambda a, bm: lax.dot_general(
            a, bm, (((1,), (1,)), ((), ())),
            preferred_element_type=jnp.float32)

        x2 = evs_ref[...].reshape(BB * 73 // 2, 128)
        ev2s = [x2[j * BB * 8:(j + 1) * BB * 8, :] for j in range(4)]
        ev1s = [x2[BB * 32 + j * BB:BB * 32 + (j + 1) * BB, :] for j in range(4)]
        ev0p = x2[BB * 36:BB * 36 + BB // 2, :]

        u128 = u_ref[...]
        odd = (uid_ref[...] & 1) == 1
        uu = jnp.where(odd, u128[:, D:], u128[:, :D])
        relm = rel_ref[...]
        P = hp(uu, relm) * (1.0 / D)
        E = jnp.exp(P)
        r0v = r0_ref[...]
        r1v = r1_ref[...]
        e0 = jnp.take_along_axis(E, r0v, axis=1)
        w0 = e0 / jnp.sum(e0, axis=1, keepdims=True)
        E2 = jnp.broadcast_to(E[:, None, :], (BB, K, N_REL)).reshape(BB * K, N_REL)
        e1v = jnp.take_along_axis(E2, r1v, axis=1)
        w1 = e1v / jnp.sum(e1v, axis=1, keepdims=True)

        li8 = lax.broadcasted_iota(jnp.int32, (K, 128), 1)
        ri8 = lax.broadcasted_iota(jnp.int32, (K, 128), 0)

        def pair_w(w, j, n):
            sel = (ri8 == jnp.where(li8 < D, 2 * j, 2 * j + 1)).astype(jnp.float32)
            return lax.dot_general(w, sel, (((1,), (0,)), ((), ())),
                                   preferred_element_type=jnp.float32)

        Rm = pair_w(w1, 0, BB * 8) * ev2s[0]
        for j in range(1, 4):
            Rm = Rm + pair_w(w1, j, BB * 8) * ev2s[j]

        Wm0 = w0_ref[...]
        bb0 = b0_ref[...]
        Wm1 = w1_ref[...]
        bb1 = b1_ref[...]
        z = jnp.zeros((D, D), jnp.float32)
        W0cat = jnp.concatenate([Wm0, Wm0], axis=1) * 0.125
        Y = hp(Rm, W0cat)
        Y3 = Y.reshape(BB, 8, D)
        BD0 = jnp.concatenate(
            [jnp.concatenate([Wm0, z], axis=1),
             jnp.concatenate([z, Wm0], axis=1)], axis=0)
        b0p = jnp.concatenate([bb0, bb0], axis=1)

        S3 = pair_w(w0, 0, BB) * ev1s[0]
        T = None
        h1s = []
        for j in range(4):
            pyj = jnp.concatenate([Y3[:, 2 * j, :], Y3[:, 2 * j + 1, :]], axis=1)
            h1j = jax.nn.relu(hp(ev1s[j], BD0) + pyj + b0p)
            h1s.append(h1j)
            tj = pair_w(w0, j, BB) * h1j
            T = tj if T is None else T + tj
            if j > 0:
                S3 = S3 + pair_w(w0, j, BB) * ev1s[j]
        aggf = (T[:, :D] + T[:, D:]) * 0.125
        agg0 = (S3[:, :D] + S3[:, D:]) * 0.125

        ev0 = jnp.stack([ev0p[:, :D], ev0p[:, D:]],
                        axis=1).reshape(BB, D)
        h0 = jax.nn.relu(hp(ev0 + agg0, Wm0) + bb0)
        i_emb = jnp.tanh(hp(h0 + aggf, Wm1) + bb1)
        out_ref[...] = jnp.sum(uu * i_emb, axis=1, keepdims=True)

    out = pl.pallas_call(
        body,
        grid=(nblk,),
        in_specs=[
            pl.BlockSpec((BB, 128), lambda i: (i, 0)),
            pl.BlockSpec((BB, 1), lambda i: (i, 0)),
            pl.BlockSpec((BP, 8, 128), lambda i: (i, 0, 0)),
            pl.BlockSpec((BB, K), lambda i: (i, 0)),
            pl.BlockSpec((BB * 8, K), lambda i: (i, 0)),
            pl.BlockSpec((N_REL, D), lambda i: (0, 0)),
            pl.BlockSpec((D, D), lambda i: (0, 0)),
            pl.BlockSpec((1, D), lambda i: (0, 0)),
            pl.BlockSpec((D, D), lambda i: (0, 0)),
            pl.BlockSpec((1, D), lambda i: (0, 0)),
        ],
        out_specs=pl.BlockSpec((BB, 1), lambda i: (i, 0)),
        out_shape=jax.ShapeDtypeStruct((Bh, 1), jnp.float32),
    )(u, uid, evs3, r0, r1, rel, W0, b0, W1, b1)
    return out.reshape(Bh)


def kernel(user_ids, item_ids, adj_entity, adj_relation, user_emb,
           entity_emb, relation_emb, W0, b0, W1, b1):
    item_ids = item_ids.astype(jnp.int32)
    user_ids = user_ids.astype(jnp.int32)
    fused_adj = jnp.concatenate(
        [adj_entity.astype(jnp.int32), adj_relation.astype(jnp.int32)], axis=1)

    user_pairs = user_emb.reshape(user_emb.shape[0] // 2, 2 * D)
    er1, u = _sc_gather_multi(
        [(fused_adj, item_ids), (user_pairs, user_ids >> 1)])
    e1 = er1[:, :K].reshape(-1)
    r0 = er1[:, K:]

    er2, = _sc_gather_multi([(fused_adj, e1)])
    e2 = er2[:, :K].reshape(-1)
    r1 = er2[:, K:]

    nblk = B // BB
    e2o = e2.reshape(nblk, BB, 8, 4, 2).transpose(0, 3, 1, 2, 4).reshape(nblk, BB * 64)
    e1o = e1.reshape(nblk, BB, 4, 2).transpose(0, 2, 1, 3).reshape(nblk, BB * 8)
    all_idx = jnp.concatenate(
        [e2o, e1o, item_ids.reshape(nblk, BB)], axis=1)

    Bh = B // 4
    nh = nblk // 4
    uid = user_ids.reshape(B, 1)
    outs = []
    for h in range(4):
        idx_h = all_idx[h * nh:(h + 1) * nh].reshape(-1)
        evs, = _sc_gather_multi([(entity_emb, idx_h)])
        evs3 = evs.reshape(Bh * 73 // 16, 8, 128)
        sl = slice(h * Bh, (h + 1) * Bh)
        sl8 = slice(h * Bh * 8, (h + 1) * Bh * 8)
        outs.append(_tc_dense(u[sl], uid[sl], evs3, r0[sl], r1[sl8],
                              relation_emb, W0, b0.reshape(1, D), W1,
                              b1.reshape(1, D), Bh=Bh))
    return jnp.concatenate(outs)
